# baseline (speedup 1.0000x reference)
import jax
import jax.numpy as jnp
from jax import lax
from jax.experimental import pallas as pl
from jax.experimental.pallas import tpu as pltpu
from jax.experimental.pallas import tpu_sc as plsc
import functools


_ROWS = 128
_COLS = 32768
_BLOCK_ROWS = 32

_N = _ROWS * _COLS
_NC = 2
_NS = 16
_NW = _NC * _NS
_PER_W = _N // _NW
_ZB = 8192
_CHUNKS = _PER_W // _ZB


def _season_block(x_ref, season_ref):
    x = x_ref[...]
    row = jax.lax.broadcasted_iota(jnp.int32, x.shape, 0)
    is_row0 = (row + pl.program_id(0) * _BLOCK_ROWS) == 0
    season_ref[...] = jnp.where(is_row0, jnp.zeros_like(x), x)


def _season_tc(x):
    grid = (_ROWS // _BLOCK_ROWS,)
    spec = pl.BlockSpec((_BLOCK_ROWS, _COLS), lambda i: (i, 0))
    return pl.pallas_call(
        _season_block,
        grid=grid,
        in_specs=[spec],
        out_specs=spec,
        out_shape=jax.ShapeDtypeStruct((_ROWS, _COLS), x.dtype),
        compiler_params=pltpu.CompilerParams(
            dimension_semantics=("parallel",),
        ),
    )(x)


@functools.partial(
    pl.kernel,
    out_type=jax.ShapeDtypeStruct((_N,), jnp.float32),
    mesh=plsc.VectorSubcoreMesh(core_axis_name="c", subcore_axis_name="s"),
    scratch_types=[
        pltpu.VMEM((_ZB,), jnp.float32),
        pltpu.SemaphoreType.DMA((_CHUNKS,)),
    ],
)
def _trend_sc(x_ref, out_ref, zbuf, sems):
    wid = lax.axis_index("s") * _NC + lax.axis_index("c")
    base = wid * _PER_W

    def _zero_step(i, carry):
        zbuf[pl.ds(i * 16, 16)] = jnp.zeros((16,), jnp.float32)
        return carry

    lax.fori_loop(0, _ZB // 16, _zero_step, 0)

    copies = [
        pltpu.async_copy(
            zbuf, out_ref.at[pl.ds(base + k * _ZB, _ZB)], sems.at[k])
        for k in range(_CHUNKS)
    ]
    for c in copies:
        c.wait()

    @pl.when(wid == 0)
    def _():
        pltpu.sync_copy(x_ref.at[pl.ds(0, _COLS)],
                        out_ref.at[pl.ds(0, _COLS)])


def kernel(x):
    trend = _trend_sc(x.reshape(_N)).reshape(_ROWS, _COLS)
    season = _season_tc(x)
    return (season, trend)


# speedup vs baseline: 2.0251x; 2.0251x over previous
import jax
import jax.numpy as jnp
from jax import lax
from jax.experimental import pallas as pl
from jax.experimental.pallas import tpu as pltpu
from jax.experimental.pallas import tpu_sc as plsc
import functools


_ROWS = 128
_COLS = 32768
_BLOCK_ROWS = 32

_NC = 2
_NS = 16
_NW = _NC * _NS
_WROWS = 8
_HALF = _COLS // 2
_ZCOLS = 2048
_CHUNKS = _HALF // _ZCOLS


def _season_block(x_ref, season_ref):
    x = x_ref[...]
    row = jax.lax.broadcasted_iota(jnp.int32, x.shape, 0)
    is_row0 = (row + pl.program_id(0) * _BLOCK_ROWS) == 0
    season_ref[...] = jnp.where(is_row0, jnp.zeros_like(x), x)


def _season_tc(x):
    grid = (_ROWS // _BLOCK_ROWS,)
    spec = pl.BlockSpec((_BLOCK_ROWS, _COLS), lambda i: (i, 0))
    return pl.pallas_call(
        _season_block,
        grid=grid,
        in_specs=[spec],
        out_specs=spec,
        out_shape=jax.ShapeDtypeStruct((_ROWS, _COLS), x.dtype),
        compiler_params=pltpu.CompilerParams(
            dimension_semantics=("parallel",),
        ),
    )(x)


@functools.partial(
    pl.kernel,
    out_type=jax.ShapeDtypeStruct((_ROWS, _COLS), jnp.float32),
    mesh=plsc.VectorSubcoreMesh(core_axis_name="c", subcore_axis_name="s"),
    scratch_types=[
        pltpu.VMEM((_WROWS, _ZCOLS), jnp.float32),
        pltpu.SemaphoreType.DMA((_CHUNKS,)),
    ],
)
def _trend_sc(x_ref, out_ref, zbuf, sems):
    wid = lax.axis_index("s") * _NC + lax.axis_index("c")
    row0 = (wid // 2) * _WROWS
    col0 = (wid % 2) * _HALF

    def _zero_step(i, carry):
        zbuf[i // (_ZCOLS // 16), pl.ds((i % (_ZCOLS // 16)) * 16, 16)] = (
            jnp.zeros((16,), jnp.float32))
        return carry

    lax.fori_loop(0, _WROWS * _ZCOLS // 16, _zero_step, 0)

    copies = [
        pltpu.async_copy(
            zbuf,
            out_ref.at[pl.ds(row0, _WROWS), pl.ds(col0 + k * _ZCOLS, _ZCOLS)],
            sems.at[k])
        for k in range(_CHUNKS)
    ]
    for c in copies:
        c.wait()

    @pl.when(row0 == 0)
    def _():
        pltpu.sync_copy(x_ref.at[pl.ds(0, 1), pl.ds(col0, _HALF)],
                        out_ref.at[pl.ds(0, 1), pl.ds(col0, _HALF)])


def kernel(x):
    trend = _trend_sc(x)
    season = _season_tc(x)
    return (season, trend)


# speedup vs baseline: 4.4689x; 2.2067x over previous
import jax
import jax.numpy as jnp
from jax.experimental import pallas as pl
from jax.experimental.pallas import tpu as pltpu


_ROWS = 128
_COLS = 32768
_BLOCK_COLS = 8192


def _decomp_block(x_ref, season_ref, trend_ref):
    x = x_ref[...]
    row = jax.lax.broadcasted_iota(jnp.int32, x.shape, 0)
    is_row0 = row == 0
    zero = jnp.zeros_like(x)
    season_ref[...] = jnp.where(is_row0, zero, x)
    trend_ref[...] = jnp.where(is_row0, x, zero)


def kernel(x):
    grid = (_COLS // _BLOCK_COLS,)
    spec = pl.BlockSpec((_ROWS, _BLOCK_COLS), lambda j: (0, j))
    season, trend = pl.pallas_call(
        _decomp_block,
        grid=grid,
        in_specs=[spec],
        out_specs=[spec, spec],
        out_shape=[
            jax.ShapeDtypeStruct((_ROWS, _COLS), x.dtype),
            jax.ShapeDtypeStruct((_ROWS, _COLS), x.dtype),
        ],
        compiler_params=pltpu.CompilerParams(
            dimension_semantics=("parallel",),
        ),
    )(x)
    return (season, trend)
